# Initial kernel scaffold; baseline (speedup 1.0000x reference)
import jax
import jax.numpy as jnp
from jax.experimental import pallas as pl


def kernel(x, edges, Wq, Wk, Wv, Wo, gamma, beta):
    raise NotImplementedError("write your pallas kernel here")


# speedup vs baseline: 9.4791x; 9.4791x over previous
import functools

import jax
import jax.numpy as jnp
from jax.experimental import pallas as pl

B, N, CIN, COUT, H, KNN = 4, 1024, 1024, 1024, 16, 16
HD = COUT // H
CG = 128
NG = COUT // CG
RB = 256


def _proj_kernel(x_ref, wq_ref, wk_ref, wv_ref, q_ref, k_ref, v_ref):
    xb = x_ref[0]
    q_ref[0] = jnp.dot(xb, wq_ref[...], preferred_element_type=jnp.float32)
    k_ref[0] = jnp.dot(xb, wk_ref[...], preferred_element_type=jnp.float32)
    v_ref[0] = jnp.dot(xb, wv_ref[...], preferred_element_type=jnp.float32)


def _count_kernel(e_ref, c_ref):
    e = e_ref[0]
    iota = jax.lax.broadcasted_iota(jnp.int32, (RB, N), 1)
    acc = jnp.zeros((RB, N), jnp.float32)
    for k in range(KNN):
        acc = acc + (e[:, k][:, None] == iota).astype(jnp.float32)
    c_ref[0] = acc


def _attn_kernel(q_ref, k_ref, v_ref, c_ref, o_ref):
    c = c_ref[0]
    mask = c > 0.0
    for h2 in range(CG // HD):
        q = q_ref[0][:, h2 * HD:(h2 + 1) * HD]
        k = k_ref[0][:, h2 * HD:(h2 + 1) * HD]
        v = v_ref[0][:, h2 * HD:(h2 + 1) * HD]
        s = jax.lax.dot_general(q, k, (((1,), (1,)), ((), ())),
                                preferred_element_type=jnp.float32)
        s = s * (1.0 / (HD ** 0.5))
        m = jnp.max(jnp.where(mask, s, -1e30), axis=1, keepdims=True)
        p = jnp.where(mask, c * jnp.exp(s - m), 0.0)
        z = jnp.sum(p, axis=1, keepdims=True)
        w = p / z
        o_ref[0, :, h2 * HD:(h2 + 1) * HD] = jnp.dot(
            w, v, preferred_element_type=jnp.float32)


def _outproj_kernel(a_ref, wo_ref, y_ref, st_ref):
    y = jnp.dot(a_ref[0], wo_ref[...], preferred_element_type=jnp.float32)
    y_ref[0] = y
    b = pl.program_id(0)

    @pl.when(b == 0)
    def _():
        st_ref[...] = jnp.zeros_like(st_ref)

    st_ref[0, :] += jnp.sum(y, axis=0)
    st_ref[1, :] += jnp.sum(y * y, axis=0)


def _bn_kernel(y_ref, st_ref, g_ref, b_ref, o_ref):
    cnt = float(B * N)
    mean = st_ref[0, :] / cnt
    var = st_ref[1, :] / cnt - mean * mean
    inv = jax.lax.rsqrt(var + 1e-5)
    scale = g_ref[0] * inv
    shift = b_ref[0] - mean * scale
    yn = y_ref[0] * scale[None, :] + shift[None, :]
    yn = jnp.where(yn >= 0.0, yn, 0.2 * yn)
    o_ref[0] = yn.T


@jax.jit
def kernel(x, edges, Wq, Wk, Wv, Wo, gamma, beta):
    xT = jnp.transpose(x, (0, 2, 1))
    e32 = edges.astype(jnp.int32)
    WqT, WkT, WvT, WoT = Wq.T, Wk.T, Wv.T, Wo.T

    qkv = pl.pallas_call(
        _proj_kernel,
        grid=(B, NG),
        in_specs=[
            pl.BlockSpec((1, N, CIN), lambda b, g: (b, 0, 0)),
            pl.BlockSpec((CIN, CG), lambda b, g: (0, g)),
            pl.BlockSpec((CIN, CG), lambda b, g: (0, g)),
            pl.BlockSpec((CIN, CG), lambda b, g: (0, g)),
        ],
        out_specs=[
            pl.BlockSpec((1, N, CG), lambda b, g: (b, 0, g)),
            pl.BlockSpec((1, N, CG), lambda b, g: (b, 0, g)),
            pl.BlockSpec((1, N, CG), lambda b, g: (b, 0, g)),
        ],
        out_shape=[jax.ShapeDtypeStruct((B, N, COUT), jnp.float32)] * 3,
    )(xT, WqT, WkT, WvT)
    QT, KT, VT = qkv

    C = pl.pallas_call(
        _count_kernel,
        grid=(B, N // RB),
        in_specs=[pl.BlockSpec((1, RB, KNN), lambda b, r: (b, r, 0))],
        out_specs=pl.BlockSpec((1, RB, N), lambda b, r: (b, r, 0)),
        out_shape=jax.ShapeDtypeStruct((B, N, N), jnp.float32),
    )(e32)

    ATT = pl.pallas_call(
        _attn_kernel,
        grid=(B, NG),
        in_specs=[
            pl.BlockSpec((1, N, CG), lambda b, g: (b, 0, g)),
            pl.BlockSpec((1, N, CG), lambda b, g: (b, 0, g)),
            pl.BlockSpec((1, N, CG), lambda b, g: (b, 0, g)),
            pl.BlockSpec((1, N, N), lambda b, g: (b, 0, 0)),
        ],
        out_specs=pl.BlockSpec((1, N, CG), lambda b, g: (b, 0, g)),
        out_shape=jax.ShapeDtypeStruct((B, N, COUT), jnp.float32),
    )(QT, KT, VT, C)

    Y, ST = pl.pallas_call(
        _outproj_kernel,
        grid=(B,),
        in_specs=[
            pl.BlockSpec((1, N, COUT), lambda b: (b, 0, 0)),
            pl.BlockSpec((COUT, COUT), lambda b: (0, 0)),
        ],
        out_specs=[
            pl.BlockSpec((1, N, COUT), lambda b: (b, 0, 0)),
            pl.BlockSpec((2, COUT), lambda b: (0, 0)),
        ],
        out_shape=[
            jax.ShapeDtypeStruct((B, N, COUT), jnp.float32),
            jax.ShapeDtypeStruct((2, COUT), jnp.float32),
        ],
    )(ATT, WoT)

    out = pl.pallas_call(
        _bn_kernel,
        grid=(B,),
        in_specs=[
            pl.BlockSpec((1, N, COUT), lambda b: (b, 0, 0)),
            pl.BlockSpec((2, COUT), lambda b: (0, 0)),
            pl.BlockSpec((1, COUT), lambda b: (0, 0)),
            pl.BlockSpec((1, COUT), lambda b: (0, 0)),
        ],
        out_specs=pl.BlockSpec((1, COUT, N), lambda b: (b, 0, 0)),
        out_shape=jax.ShapeDtypeStruct((B, COUT, N), jnp.float32),
    )(Y, ST, gamma[None, :], beta[None, :])
    return out


# speedup vs baseline: 16.6005x; 1.7513x over previous
import jax
import jax.numpy as jnp
from jax.experimental import pallas as pl

B, N, CIN, COUT, H, KNN = 4, 1024, 1024, 1024, 16, 16
HD = COUT // H
CG = 128
NG = COUT // CG
RB = 256
NHG = CG // HD


def _count_kernel(e_ref, c_ref):
    e = e_ref[0]
    r = pl.program_id(1)
    iota = r * RB + jax.lax.broadcasted_iota(jnp.int32, (RB, N), 0)
    acc = jnp.zeros((RB, N), jnp.float32)
    for k in range(KNN):
        acc = acc + (e[k:k + 1, :] == iota).astype(jnp.float32)
    c_ref[0] = acc


def _attn_kernel(x_ref, xt_ref, wq_ref, wkt_ref, wv_ref, c_ref, o_ref):
    xb = x_ref[0]
    xtb = xt_ref[0]
    qT = jnp.dot(wq_ref[...], xb, preferred_element_type=jnp.float32)
    kR = jnp.dot(xtb, wkt_ref[...], preferred_element_type=jnp.float32)
    vT = jnp.dot(wv_ref[...], xb, preferred_element_type=jnp.float32)
    c = c_ref[0]
    for h in range(NHG):
        sl = slice(h * HD, (h + 1) * HD)
        sT = jnp.dot(kR[:, sl], qT[sl, :],
                     preferred_element_type=jnp.float32)
        p = c * jnp.exp(jnp.minimum(sT, 60.0))
        z = jnp.sum(p, axis=0, keepdims=True)
        a = jnp.dot(vT[sl, :], p, preferred_element_type=jnp.float32)
        o_ref[0, sl, :] = a / z


def _outproj_kernel(a_ref, wo_ref, y_ref, st_ref):
    y = jnp.dot(wo_ref[...], a_ref[0], preferred_element_type=jnp.float32)
    y_ref[0] = y
    b = pl.program_id(0)

    @pl.when(b == 0)
    def _():
        st_ref[...] = jnp.zeros_like(st_ref)

    st_ref[:, 0:1] += jnp.sum(y, axis=1, keepdims=True)
    st_ref[:, 1:2] += jnp.sum(y * y, axis=1, keepdims=True)


def _bn_kernel(y_ref, st_ref, g_ref, b_ref, o_ref):
    cnt = float(B * N)
    mean = st_ref[:, 0:1] / cnt
    var = st_ref[:, 1:2] / cnt - mean * mean
    inv = jax.lax.rsqrt(var + 1e-5)
    scale = g_ref[...] * inv
    shift = b_ref[...] - mean * scale
    yn = y_ref[0] * scale + shift
    o_ref[0] = jnp.where(yn >= 0.0, yn, 0.2 * yn)


@jax.jit
def kernel(x, edges, Wq, Wk, Wv, Wo, gamma, beta):
    xT = jnp.transpose(x, (0, 2, 1))
    eT = jnp.transpose(edges.astype(jnp.int32), (0, 2, 1))
    Wq_s = Wq * jnp.float32(1.0 / (HD ** 0.5))
    WkT = Wk.T

    C = pl.pallas_call(
        _count_kernel,
        grid=(B, N // RB),
        in_specs=[pl.BlockSpec((1, KNN, N), lambda b, r: (b, 0, 0))],
        out_specs=pl.BlockSpec((1, RB, N), lambda b, r: (b, r, 0)),
        out_shape=jax.ShapeDtypeStruct((B, N, N), jnp.float32),
    )(eT)

    ATT = pl.pallas_call(
        _attn_kernel,
        grid=(B, NG),
        in_specs=[
            pl.BlockSpec((1, CIN, N), lambda b, g: (b, 0, 0)),
            pl.BlockSpec((1, N, CIN), lambda b, g: (b, 0, 0)),
            pl.BlockSpec((CG, CIN), lambda b, g: (g, 0)),
            pl.BlockSpec((CIN, CG), lambda b, g: (0, g)),
            pl.BlockSpec((CG, CIN), lambda b, g: (g, 0)),
            pl.BlockSpec((1, N, N), lambda b, g: (b, 0, 0)),
        ],
        out_specs=pl.BlockSpec((1, CG, N), lambda b, g: (b, g, 0)),
        out_shape=jax.ShapeDtypeStruct((B, COUT, N), jnp.float32),
    )(x, xT, Wq_s, WkT, Wv, C)

    Y, ST = pl.pallas_call(
        _outproj_kernel,
        grid=(B,),
        in_specs=[
            pl.BlockSpec((1, COUT, N), lambda b: (b, 0, 0)),
            pl.BlockSpec((COUT, COUT), lambda b: (0, 0)),
        ],
        out_specs=[
            pl.BlockSpec((1, COUT, N), lambda b: (b, 0, 0)),
            pl.BlockSpec((COUT, 2), lambda b: (0, 0)),
        ],
        out_shape=[
            jax.ShapeDtypeStruct((B, COUT, N), jnp.float32),
            jax.ShapeDtypeStruct((COUT, 2), jnp.float32),
        ],
    )(ATT, Wo)

    out = pl.pallas_call(
        _bn_kernel,
        grid=(B,),
        in_specs=[
            pl.BlockSpec((1, COUT, N), lambda b: (b, 0, 0)),
            pl.BlockSpec((COUT, 2), lambda b: (0, 0)),
            pl.BlockSpec((COUT, 1), lambda b: (0, 0)),
            pl.BlockSpec((COUT, 1), lambda b: (0, 0)),
        ],
        out_specs=pl.BlockSpec((1, COUT, N), lambda b: (b, 0, 0)),
        out_shape=jax.ShapeDtypeStruct((B, COUT, N), jnp.float32),
    )(Y, ST, gamma[:, None], beta[:, None])
    return out
